# baseline (speedup 1.0000x reference)
import functools

import jax
import jax.numpy as jnp
from jax import lax
from jax.experimental import pallas as pl
from jax.experimental.pallas import tpu as pltpu
from jax.experimental.pallas import tpu_sc as plsc

L = 16
UNROLL = 8
S_SC = 896
TC_BS = 128


def _sc_pe_add(B, S_sc, D, s_off):
    NC, NS = 2, 16
    NW = NC * NS
    sw = S_sc // NW
    K = sw // 2

    mesh = plsc.VectorSubcoreMesh(core_axis_name="c", subcore_axis_name="s")

    @functools.partial(
        pl.kernel,
        out_type=jax.ShapeDtypeStruct((B, S_sc, D), jnp.float32),
        mesh=mesh,
        scratch_types=(
            [pltpu.VMEM((B, D), jnp.float32) for _ in range(2)]
            + [pltpu.VMEM((B, D), jnp.float32) for _ in range(2)]
            + [pltpu.VMEM((1, D), jnp.float32) for _ in range(2)]
            + [pltpu.SemaphoreType.DMA for _ in range(6)]
        ),
    )
    def body(x_hbm, pe_hbm, out_hbm, *scratch):
        xa = scratch[0:2]
        ov = scratch[2:4]
        pe_v = scratch[4:6]
        sem_ld = scratch[6:8]
        sem_pe = scratch[8:10]
        sem_st = scratch[10:12]

        wid = lax.axis_index("s") * NC + lax.axis_index("c")
        base = wid * sw

        def issue_loads(jj, sl):
            sg = s_off + sl
            pltpu.async_copy(pe_hbm.at[pl.ds(sg, 1)], pe_v[jj], sem_pe[jj])
            for b in range(B):
                pltpu.async_copy(
                    x_hbm.at[b, pl.ds(sg, 1)],
                    xa[jj].at[pl.ds(b, 1)],
                    sem_ld[jj],
                )

        issue_loads(0, base)
        issue_loads(1, base + 1)

        def step(k, carry):
            for jj in range(2):
                sl = base + 2 * k + jj
                sg = s_off + sl
                pltpu.make_async_copy(
                    pe_hbm.at[pl.ds(sg, 1)], pe_v[jj], sem_pe[jj]
                ).wait()
                for b in range(B):
                    pltpu.make_async_copy(
                        x_hbm.at[b, pl.ds(sg, 1)],
                        xa[jj].at[pl.ds(b, 1)],
                        sem_ld[jj],
                    ).wait()

                @pl.when(k > 0)
                def _(jj=jj, sl=sl):
                    for b in range(B):
                        pltpu.make_async_copy(
                            ov[jj].at[pl.ds(b, 1)],
                            out_hbm.at[b, pl.ds(sl, 1)],
                            sem_st[jj],
                        ).wait()

                @plsc.parallel_loop(0, D // L, unroll=UNROLL)
                def cbody(i, jj=jj):
                    off = i * L
                    p = pe_v[jj][0, pl.ds(off, L)]
                    for b in range(B):
                        ov[jj][b, pl.ds(off, L)] = xa[jj][b, pl.ds(off, L)] + p

                for b in range(B):
                    pltpu.async_copy(
                        ov[jj].at[pl.ds(b, 1)],
                        out_hbm.at[b, pl.ds(sl, 1)],
                        sem_st[jj],
                    )

                @pl.when(k < K - 1)
                def _(jj=jj, sl=sl):
                    issue_loads(jj, sl + 2)
            return carry

        lax.fori_loop(0, K, step, 0)

        for jj in range(2):
            sl = base + sw - 2 + jj
            for b in range(B):
                pltpu.make_async_copy(
                    ov[jj].at[pl.ds(b, 1)],
                    out_hbm.at[b, pl.ds(sl, 1)],
                    sem_st[jj],
                ).wait()

    return body


def _tc_add_body(x_ref, pe_ref, o_ref):
    o_ref[...] = x_ref[...] + pe_ref[...]


def kernel(x, pe_table):
    B, S, D = x.shape
    s_tc = S - S_SC

    sc_part = _sc_pe_add(B, S_SC, D, s_tc)(x, pe_table)

    tc_out = pl.pallas_call(
        _tc_add_body,
        grid=(B, s_tc // TC_BS),
        in_specs=[
            pl.BlockSpec((1, TC_BS, D), lambda b, i: (b, i, 0)),
            pl.BlockSpec((TC_BS, D), lambda b, i: (i, 0)),
        ],
        out_specs=pl.BlockSpec((1, TC_BS, D), lambda b, i: (b, i, 0)),
        out_shape=jax.ShapeDtypeStruct((B, S, D), x.dtype),
    )(x, pe_table)

    return lax.dynamic_update_slice(tc_out, sc_part, (0, s_tc, 0))


# speedup vs baseline: 1.3458x; 1.3458x over previous
import functools

import jax
import jax.numpy as jnp
from jax import lax
from jax.experimental import pallas as pl
from jax.experimental.pallas import tpu as pltpu
from jax.experimental.pallas import tpu_sc as plsc

L = 16
UNROLL = 8


def _sc_pe_add(B, S_sc, D, s_off):
    NC, NS = 2, 16
    NW = NC * NS
    sw = S_sc // NW
    K = sw // 2

    mesh = plsc.VectorSubcoreMesh(core_axis_name="c", subcore_axis_name="s")

    @functools.partial(
        pl.kernel,
        out_type=jax.ShapeDtypeStruct((B, S_sc, D), jnp.float32),
        mesh=mesh,
        scratch_types=(
            [pltpu.VMEM((B, 1, D), jnp.float32) for _ in range(2)]
            + [pltpu.VMEM((B, 1, D), jnp.float32) for _ in range(2)]
            + [pltpu.VMEM((1, D), jnp.float32) for _ in range(2)]
            + [pltpu.SemaphoreType.DMA for _ in range(6)]
        ),
    )
    def body(x_hbm, pe_hbm, out_hbm, *scratch):
        xa = scratch[0:2]
        ov = scratch[2:4]
        pe_v = scratch[4:6]
        sem_ld = scratch[6:8]
        sem_pe = scratch[8:10]
        sem_st = scratch[10:12]

        wid = lax.axis_index("s") * NC + lax.axis_index("c")
        base = wid * sw

        def issue_loads(jj, sl):
            sg = s_off + sl
            pltpu.async_copy(pe_hbm.at[pl.ds(sg, 1)], pe_v[jj], sem_pe[jj])
            pltpu.async_copy(x_hbm.at[:, pl.ds(sg, 1)], xa[jj], sem_ld[jj])

        issue_loads(0, base)
        issue_loads(1, base + 1)

        def step(k, carry):
            for jj in range(2):
                sl = base + 2 * k + jj
                sg = s_off + sl
                pltpu.make_async_copy(
                    pe_hbm.at[pl.ds(sg, 1)], pe_v[jj], sem_pe[jj]
                ).wait()
                pltpu.make_async_copy(
                    x_hbm.at[:, pl.ds(sg, 1)], xa[jj], sem_ld[jj]
                ).wait()

                @pl.when(k > 0)
                def _(jj=jj, sl=sl):
                    pltpu.make_async_copy(
                        ov[jj], out_hbm.at[:, pl.ds(sl, 1)], sem_st[jj]
                    ).wait()

                @plsc.parallel_loop(0, D // L, unroll=UNROLL)
                def cbody(i, jj=jj):
                    off = i * L
                    p = pe_v[jj][0, pl.ds(off, L)]
                    for b in range(B):
                        ov[jj][b, 0, pl.ds(off, L)] = (
                            xa[jj][b, 0, pl.ds(off, L)] + p
                        )

                pltpu.async_copy(
                    ov[jj], out_hbm.at[:, pl.ds(sl, 1)], sem_st[jj]
                )

                @pl.when(k < K - 1)
                def _(jj=jj, sl=sl):
                    issue_loads(jj, sl + 2)
            return carry

        lax.fori_loop(0, K, step, 0)

        for jj in range(2):
            sl = base + sw - 2 + jj
            pltpu.make_async_copy(
                ov[jj], out_hbm.at[:, pl.ds(sl, 1)], sem_st[jj]
            ).wait()

    return body


def kernel(x, pe_table):
    B, S, D = x.shape
    return _sc_pe_add(B, S, D, 0)(x, pe_table)
